# baseline (speedup 1.0000x reference)
import functools

import jax
import jax.numpy as jnp
from jax import lax
from jax.experimental import pallas as pl
from jax.experimental.pallas import tpu as pltpu
from jax.experimental.pallas import tpu_sc as plsc

_B = 16384
_D = 64
_L = 16
_NC = 2
_NS = 16
_NW = _NC * _NS
_BPW = _B // _NW
_CHUNK = 128
_NCHUNK = _BPW // _CHUNK
_NBLK = _D // _L


def _sc_gather_score(users, pos, neg, t0, t1, t2, item):
    mesh = plsc.VectorSubcoreMesh(core_axis_name="c", subcore_axis_name="s")

    @functools.partial(
        pl.kernel,
        out_type=(
            jax.ShapeDtypeStruct((_B, _L), jnp.float32),
            jax.ShapeDtypeStruct((_NW, _L), jnp.float32),
        ),
        mesh=mesh,
        compiler_params=pltpu.CompilerParams(use_tc_tiling_on_sc=False),
        scratch_types=[
            pltpu.VMEM((_BPW,), jnp.int32),
            pltpu.VMEM((_BPW,), jnp.int32),
            pltpu.VMEM((_BPW,), jnp.int32),
            pltpu.VMEM((_CHUNK, _D), jnp.float32),
            pltpu.VMEM((_CHUNK, _D), jnp.float32),
            pltpu.VMEM((_CHUNK, _D), jnp.float32),
            pltpu.VMEM((_CHUNK, _D), jnp.float32),
            pltpu.VMEM((_CHUNK, _D), jnp.float32),
            pltpu.VMEM((_BPW, _L), jnp.float32),
            pltpu.VMEM((_L,), jnp.float32),
            pltpu.SemaphoreType.DMA,
        ],
    )
    def sc_body(users_h, pos_h, neg_h, t0_h, t1_h, t2_h, item_h,
                q_h, r_h,
                uidx, pidx, nidx, t0v, t1v, t2v, pv, nv, qv, regv, sem):
        wid = lax.axis_index("s") * _NC + lax.axis_index("c")
        base = wid * _BPW
        pltpu.sync_copy(users_h.at[pl.ds(base, _BPW)], uidx)
        pltpu.sync_copy(pos_h.at[pl.ds(base, _BPW)], pidx)
        pltpu.sync_copy(neg_h.at[pl.ds(base, _BPW)], nidx)

        def chunk_body(ci, regacc):
            off = ci * _CHUNK
            isl = pl.ds(off, _CHUNK)
            c0 = pltpu.async_copy(t0_h.at[uidx.at[isl]], t0v, sem)
            c1 = pltpu.async_copy(t1_h.at[uidx.at[isl]], t1v, sem)
            c2 = pltpu.async_copy(t2_h.at[uidx.at[isl]], t2v, sem)
            c3 = pltpu.async_copy(item_h.at[pidx.at[isl]], pv, sem)
            c4 = pltpu.async_copy(item_h.at[nidx.at[isl]], nv, sem)
            c0.wait()
            c1.wait()
            c2.wait()
            c3.wait()
            c4.wait()

            def row_body(r, reg):
                q = None
                for k in range(_NBLK):
                    sl = pl.ds(k * _L, _L)
                    a0 = t0v[r, sl]
                    a1 = t1v[r, sl]
                    a2 = t2v[r, sl]
                    pp = pv[r, sl]
                    nn = nv[r, sl]
                    u = a0 + 0.5 * (a1 + a2)
                    term = u * (nn - pp)
                    q = term if q is None else q + term
                    reg = reg + (u * u + pp * pp + nn * nn)
                qv[off + r, :] = q
                return reg

            return lax.fori_loop(0, _CHUNK, row_body, regacc)

        regacc = lax.fori_loop(0, _NCHUNK, chunk_body,
                               jnp.zeros((_L,), jnp.float32))
        pltpu.sync_copy(qv, q_h.at[pl.ds(base, _BPW)])
        regv[...] = regacc
        pltpu.sync_copy(regv, r_h.at[wid])

    return sc_body(users, pos, neg, t0, t1, t2, item)


def _tc_finalize(q, r):
    def tc_body(q_ref, r_ref, loss_ref, reg_ref):
        d = jnp.sum(q_ref[...], axis=1)
        loss_ref[0, 0] = jnp.mean(jax.nn.softplus(d))
        reg_ref[0, 0] = 0.5 * jnp.sum(r_ref[...]) / float(_B)

    loss, reg = pl.pallas_call(
        tc_body,
        out_shape=(
            jax.ShapeDtypeStruct((1, 1), jnp.float32),
            jax.ShapeDtypeStruct((1, 1), jnp.float32),
        ),
        out_specs=(
            pl.BlockSpec(memory_space=pltpu.SMEM),
            pl.BlockSpec(memory_space=pltpu.SMEM),
        ),
    )(q, r)
    return loss[0, 0], reg[0, 0]


def kernel(users, pos, neg, user_table_0, user_table_1, user_table_2,
           item_table):
    q, r = _sc_gather_score(users, pos, neg, user_table_0, user_table_1,
                            user_table_2, item_table)
    loss, reg_loss = _tc_finalize(q, r)
    return (loss, reg_loss)


# speedup vs baseline: 1.3300x; 1.3300x over previous
import functools

import jax
import jax.numpy as jnp
from jax import lax
from jax.experimental import pallas as pl
from jax.experimental.pallas import tpu as pltpu
from jax.experimental.pallas import tpu_sc as plsc

_B = 16384
_D = 64
_L = 16
_NC = 2
_NS = 16
_NW = _NC * _NS
_BPW = _B // _NW
_CHUNK = 64
_NCHUNK = _BPW // _CHUNK
_NBLK = _D // _L


def _sc_gather_score(users, pos, neg, t0, t1, t2, item):
    mesh = plsc.VectorSubcoreMesh(core_axis_name="c", subcore_axis_name="s")

    @functools.partial(
        pl.kernel,
        out_type=(
            jax.ShapeDtypeStruct((_B, _L), jnp.float32),
            jax.ShapeDtypeStruct((_NW, _L), jnp.float32),
        ),
        mesh=mesh,
        scratch_types=[
            pltpu.VMEM((_BPW,), jnp.int32),
            pltpu.VMEM((_BPW,), jnp.int32),
            pltpu.VMEM((_BPW,), jnp.int32),
            pltpu.VMEM((_CHUNK, _D), jnp.float32),
            pltpu.VMEM((_CHUNK, _D), jnp.float32),
            pltpu.VMEM((_CHUNK, _D), jnp.float32),
            pltpu.VMEM((_CHUNK, _D), jnp.float32),
            pltpu.VMEM((_CHUNK, _D), jnp.float32),
            pltpu.VMEM((_BPW, _L), jnp.float32),
            pltpu.VMEM((_L,), jnp.float32),
            pltpu.SemaphoreType.DMA,
        ],
    )
    def sc_body(users_h, pos_h, neg_h, t0_h, t1_h, t2_h, item_h,
                q_h, r_h,
                uidx, pidx, nidx, t0v, t1v, t2v, pv, nv, qv, regv, sem):
        wid = lax.axis_index("s") * _NC + lax.axis_index("c")
        base = wid * _BPW
        pltpu.sync_copy(users_h.at[pl.ds(base, _BPW)], uidx)
        pltpu.sync_copy(pos_h.at[pl.ds(base, _BPW)], pidx)
        pltpu.sync_copy(neg_h.at[pl.ds(base, _BPW)], nidx)

        def chunk_body(ci, regacc):
            off = ci * _CHUNK
            for g in range(_CHUNK // _L):
                uvec = uidx[pl.ds(off + g * _L, _L)]
                pvec = pidx[pl.ds(off + g * _L, _L)]
                nvec = nidx[pl.ds(off + g * _L, _L)]
                for l in range(_L):
                    j = g * _L + l
                    pltpu.async_copy(t0_h.at[uvec[l]], t0v.at[j], sem)
                    pltpu.async_copy(t1_h.at[uvec[l]], t1v.at[j], sem)
                    pltpu.async_copy(t2_h.at[uvec[l]], t2v.at[j], sem)
                    pltpu.async_copy(item_h.at[pvec[l]], pv.at[j], sem)
                    pltpu.async_copy(item_h.at[nvec[l]], nv.at[j], sem)
            pltpu.make_async_copy(t0_h.at[pl.ds(0, _CHUNK)], t0v, sem).wait()
            pltpu.make_async_copy(t1_h.at[pl.ds(0, _CHUNK)], t1v, sem).wait()
            pltpu.make_async_copy(t2_h.at[pl.ds(0, _CHUNK)], t2v, sem).wait()
            pltpu.make_async_copy(item_h.at[pl.ds(0, _CHUNK)], pv, sem).wait()
            pltpu.make_async_copy(item_h.at[pl.ds(0, _CHUNK)], nv, sem).wait()

            def row_body(r, reg):
                q = None
                for k in range(_NBLK):
                    sl = pl.ds(k * _L, _L)
                    a0 = t0v[r, sl]
                    a1 = t1v[r, sl]
                    a2 = t2v[r, sl]
                    pp = pv[r, sl]
                    nn = nv[r, sl]
                    u = a0 + 0.5 * (a1 + a2)
                    term = u * (nn - pp)
                    q = term if q is None else q + term
                    reg = reg + (u * u + pp * pp + nn * nn)
                qv[off + r, :] = q
                return reg

            return lax.fori_loop(0, _CHUNK, row_body, regacc)

        regacc = lax.fori_loop(0, _NCHUNK, chunk_body,
                               jnp.zeros((_L,), jnp.float32))
        pltpu.sync_copy(qv, q_h.at[pl.ds(base, _BPW)])
        regv[...] = regacc
        pltpu.sync_copy(regv, r_h.at[wid])

    return sc_body(users, pos, neg, t0, t1, t2, item)


def _tc_finalize(q, r):
    def tc_body(q_ref, r_ref, loss_ref, reg_ref):
        d = jnp.sum(q_ref[...], axis=1)
        loss_ref[0, 0] = jnp.mean(jax.nn.softplus(d))
        reg_ref[0, 0] = 0.5 * jnp.sum(r_ref[...]) / float(_B)

    loss, reg = pl.pallas_call(
        tc_body,
        out_shape=(
            jax.ShapeDtypeStruct((1, 1), jnp.float32),
            jax.ShapeDtypeStruct((1, 1), jnp.float32),
        ),
        out_specs=(
            pl.BlockSpec(memory_space=pltpu.SMEM),
            pl.BlockSpec(memory_space=pltpu.SMEM),
        ),
    )(q, r)
    return loss[0, 0], reg[0, 0]


def kernel(users, pos, neg, user_table_0, user_table_1, user_table_2,
           item_table):
    q, r = _sc_gather_score(users, pos, neg, user_table_0, user_table_1,
                            user_table_2, item_table)
    loss, reg_loss = _tc_finalize(q, r)
    return (loss, reg_loss)
